# baseline (speedup 1.0000x reference)
import functools

import jax
import jax.numpy as jnp
from jax import lax
from jax.experimental import pallas as pl
from jax.experimental.pallas import tpu as pltpu
from jax.experimental.pallas import tpu_sc as plsc

D_MODEL = 768
N_TIERS = 8
B = 8192

NC = 2
NS = 16
NW = NC * NS
B_PER_W = B // NW
CHUNK = 32
N_CHUNKS = B_PER_W // CHUNK
N_SLICES = D_MODEL // 16

TB = 1024


def _router_body(q_ref, w_ref, logits_ref, gw_ref, idx0_ref, idx1_ref, w0_ref):
    q = q_ref[...]
    w = w_ref[...]
    lt = lax.dot_general(
        w, q, (((1,), (1,)), ((), ())), preferred_element_type=jnp.float32
    )

    iota = lax.broadcasted_iota(jnp.int32, (N_TIERS, TB), 0)
    m1 = jnp.max(lt, axis=0, keepdims=True)
    i1 = jnp.min(jnp.where(lt == m1, iota, N_TIERS), axis=0, keepdims=True)
    masked = jnp.where(iota == i1, jnp.float32(-jnp.inf), lt)
    m2 = jnp.max(masked, axis=0, keepdims=True)
    i2 = jnp.min(jnp.where(masked == m2, iota, N_TIERS), axis=0, keepdims=True)

    e = jnp.exp(m2 - m1)
    w0 = 1.0 / (1.0 + e)
    w1 = e / (1.0 + e)

    gw_t = jnp.where(iota == i1, w0, 0.0) + jnp.where(iota == i2, w1, 0.0)
    logits_ref[...] = lt.T
    gw_ref[...] = gw_t.T

    gid = pl.program_id(0) * TB + lax.broadcasted_iota(jnp.int32, (1, TB), 1)
    idx0_ref[...] = (i1 * B + gid).reshape(TB)
    idx1_ref[...] = (i2 * B + gid).reshape(TB)
    w0_ref[...] = w0.reshape(TB)


_router = pl.pallas_call(
    _router_body,
    grid=(B // TB,),
    in_specs=[
        pl.BlockSpec((TB, D_MODEL), lambda i: (i, 0)),
        pl.BlockSpec((N_TIERS, D_MODEL), lambda i: (0, 0)),
    ],
    out_specs=[
        pl.BlockSpec((TB, N_TIERS), lambda i: (i, 0)),
        pl.BlockSpec((TB, N_TIERS), lambda i: (i, 0)),
        pl.BlockSpec((TB,), lambda i: (i,)),
        pl.BlockSpec((TB,), lambda i: (i,)),
        pl.BlockSpec((TB,), lambda i: (i,)),
    ],
    out_shape=[
        jax.ShapeDtypeStruct((B, N_TIERS), jnp.float32),
        jax.ShapeDtypeStruct((B, N_TIERS), jnp.float32),
        jax.ShapeDtypeStruct((B,), jnp.int32),
        jax.ShapeDtypeStruct((B,), jnp.int32),
        jax.ShapeDtypeStruct((B,), jnp.float32),
    ],
)


def _combine_body(table, idx0, idx1, w0t, out,
                  idx0_a, idx0_b, idx0_c, idx1_a, idx1_b, idx1_c,
                  wexp_a, wexp_b, wexp_c, widx_a, widx_b, widx_c,
                  rows0_a, rows0_b, rows0_c, rows1_a, rows1_b,
                  semg_a, semg_b, semg_c, semw_a, semw_b):
    wid = lax.axis_index("s") * NC + lax.axis_index("c")
    base_w = wid * B_PER_W
    zero16 = lax.broadcasted_iota(jnp.int32, (16,), 0) * 0
    wexp_p = (wexp_a, wexp_b, wexp_c)
    widx_p = (widx_a, widx_b, widx_c)

    idx0_p = (idx0_a, idx0_b, idx0_c)
    idx1_p = (idx1_a, idx1_b, idx1_c)
    rows0_p = (rows0_a, rows0_b, rows0_c)
    rows1_p = (rows1_a, rows1_b)
    semg = (semg_a, semg_b, semg_c)
    semw = (semw_a, semw_b)

    def start_gathers(c):
        q3 = c % 3
        base = base_w + c * CHUNK
        pltpu.sync_copy(idx0.at[pl.ds(base, CHUNK)], idx0_p[q3])
        pltpu.sync_copy(idx1.at[pl.ds(base, CHUNK)], idx1_p[q3])
        widx = widx_p[q3]
        for s in range(CHUNK):
            widx[pl.ds(s * 16, 16)] = zero16 + (base + s)
        cpa = pltpu.async_copy(table.at[idx0_p[q3]], rows0_p[q3], semg[q3])
        cpb = pltpu.async_copy(table.at[idx1_p[q3]], rows1_p[c % 2], semg[q3])
        cpw = pltpu.async_copy(w0t.at[widx], wexp_p[q3], semg[q3])
        return cpa, cpb, cpw

    gathers = {0: start_gathers(0)}
    if N_CHUNKS > 1:
        gathers[1] = start_gathers(1)
    wbs = {}
    for c in range(N_CHUNKS):
        q3 = c % 3
        cpa, cpb, cpw = gathers.pop(c)
        cpa.wait()
        cpb.wait()
        cpw.wait()
        rows0 = rows0_p[q3]
        rows1 = rows1_p[c % 2]
        wexp = wexp_p[q3]

        def tok_body(t, carry):
            w = wexp[pl.ds(t * 16, 16)]
            for d in range(N_SLICES):
                sl = pl.ds(d * 16, 16)
                a = rows0[t, sl]
                b = rows1[t, sl]
                rows0[t, sl] = b + w * (a - b)
            return carry

        lax.fori_loop(0, CHUNK, tok_body, 0)
        wbs[c] = pltpu.async_copy(
            rows0, out.at[pl.ds(base_w + c * CHUNK, CHUNK)], semw[c % 2])
        if c + 2 < N_CHUNKS:
            if c - 1 in wbs:
                wbs.pop(c - 1).wait()
            gathers[c + 2] = start_gathers(c + 2)
    for c in sorted(wbs):
        wbs.pop(c).wait()


@functools.lru_cache(maxsize=1)
def _make_combine():
    return pl.kernel(
        _combine_body,
        out_type=jax.ShapeDtypeStruct((B, D_MODEL), jnp.float32),
        mesh=plsc.VectorSubcoreMesh(core_axis_name="c", subcore_axis_name="s"),
        scratch_types=[
            pltpu.VMEM((CHUNK,), jnp.int32),
            pltpu.VMEM((CHUNK,), jnp.int32),
            pltpu.VMEM((CHUNK,), jnp.int32),
            pltpu.VMEM((CHUNK,), jnp.int32),
            pltpu.VMEM((CHUNK,), jnp.int32),
            pltpu.VMEM((CHUNK,), jnp.int32),
            pltpu.VMEM((CHUNK * 16,), jnp.float32),
            pltpu.VMEM((CHUNK * 16,), jnp.float32),
            pltpu.VMEM((CHUNK * 16,), jnp.float32),
            pltpu.VMEM((CHUNK * 16,), jnp.int32),
            pltpu.VMEM((CHUNK * 16,), jnp.int32),
            pltpu.VMEM((CHUNK * 16,), jnp.int32),
            pltpu.VMEM((CHUNK, D_MODEL), jnp.float32),
            pltpu.VMEM((CHUNK, D_MODEL), jnp.float32),
            pltpu.VMEM((CHUNK, D_MODEL), jnp.float32),
            pltpu.VMEM((CHUNK, D_MODEL), jnp.float32),
            pltpu.VMEM((CHUNK, D_MODEL), jnp.float32),
            pltpu.SemaphoreType.DMA,
            pltpu.SemaphoreType.DMA,
            pltpu.SemaphoreType.DMA,
            pltpu.SemaphoreType.DMA,
            pltpu.SemaphoreType.DMA,
        ],
    )


def kernel(tier_outputs, query, W):
    logits, gate_weights, idx0, idx1, w0 = _router(query, W)
    table = tier_outputs.reshape(N_TIERS * B, D_MODEL)
    merged = _make_combine()(table, idx0, idx1, w0)
    return merged, gate_weights, logits


# speedup vs baseline: 1.2768x; 1.2768x over previous
import functools

import jax
import jax.numpy as jnp
from jax import lax
from jax.experimental import pallas as pl
from jax.experimental.pallas import tpu as pltpu
from jax.experimental.pallas import tpu_sc as plsc

D_MODEL = 768
N_TIERS = 8
B = 8192

NC = 2
NS = 16
NW = NC * NS
B_PER_W = B // NW
CHUNK = 32
N_CHUNKS = B_PER_W // CHUNK
N_SLICES = D_MODEL // 16

TB = 1024


def _router_body(q_ref, w_ref, logits_ref, gw_ref, idx0_ref, idx1_ref, w0_ref):
    q = q_ref[...]
    w = w_ref[...]
    lt = lax.dot_general(
        w, q, (((1,), (1,)), ((), ())), preferred_element_type=jnp.float32
    )

    iota = lax.broadcasted_iota(jnp.int32, (N_TIERS, TB), 0)
    m1 = jnp.max(lt, axis=0, keepdims=True)
    i1 = jnp.min(jnp.where(lt == m1, iota, N_TIERS), axis=0, keepdims=True)
    masked = jnp.where(iota == i1, jnp.float32(-jnp.inf), lt)
    m2 = jnp.max(masked, axis=0, keepdims=True)
    i2 = jnp.min(jnp.where(masked == m2, iota, N_TIERS), axis=0, keepdims=True)

    e = jnp.exp(m2 - m1)
    w0 = 1.0 / (1.0 + e)
    w1 = e / (1.0 + e)

    gw_t = jnp.where(iota == i1, w0, 0.0) + jnp.where(iota == i2, w1, 0.0)
    logits_ref[...] = lt.T
    gw_ref[...] = gw_t.T

    gid = pl.program_id(0) * TB + lax.broadcasted_iota(jnp.int32, (1, TB), 1)
    idx0_ref[...] = (i1 * B + gid).reshape(TB)
    idx1_ref[...] = (i2 * B + gid).reshape(TB)
    w0_ref[...] = w0.reshape(TB)


_router = pl.pallas_call(
    _router_body,
    grid=(B // TB,),
    in_specs=[
        pl.BlockSpec((TB, D_MODEL), lambda i: (i, 0)),
        pl.BlockSpec((N_TIERS, D_MODEL), lambda i: (0, 0)),
    ],
    out_specs=[
        pl.BlockSpec((TB, N_TIERS), lambda i: (i, 0)),
        pl.BlockSpec((TB, N_TIERS), lambda i: (i, 0)),
        pl.BlockSpec((TB,), lambda i: (i,)),
        pl.BlockSpec((TB,), lambda i: (i,)),
        pl.BlockSpec((TB,), lambda i: (i,)),
    ],
    out_shape=[
        jax.ShapeDtypeStruct((B, N_TIERS), jnp.float32),
        jax.ShapeDtypeStruct((B, N_TIERS), jnp.float32),
        jax.ShapeDtypeStruct((B,), jnp.int32),
        jax.ShapeDtypeStruct((B,), jnp.int32),
        jax.ShapeDtypeStruct((B,), jnp.float32),
    ],
)


def _combine_body(table, idx0, idx1, w0x, out,
                  idx0_a, idx0_b, idx0_c, idx1_a, idx1_b, idx1_c, w0_v,
                  rows0_a, rows0_b, rows0_c, rows1_a, rows1_b,
                  semg_a, semg_b, semg_c, semw_a, semw_b):
    wid = lax.axis_index("s") * NC + lax.axis_index("c")
    base_w = wid * B_PER_W
    pltpu.sync_copy(w0x.at[pl.ds(base_w * 16, B_PER_W * 16)], w0_v)

    idx0_p = (idx0_a, idx0_b, idx0_c)
    idx1_p = (idx1_a, idx1_b, idx1_c)
    rows0_p = (rows0_a, rows0_b, rows0_c)
    rows1_p = (rows1_a, rows1_b)
    semg = (semg_a, semg_b, semg_c)
    semw = (semw_a, semw_b)

    def start_gathers(c):
        q3 = c % 3
        base = base_w + c * CHUNK
        pltpu.sync_copy(idx0.at[pl.ds(base, CHUNK)], idx0_p[q3])
        pltpu.sync_copy(idx1.at[pl.ds(base, CHUNK)], idx1_p[q3])
        cpa = pltpu.async_copy(table.at[idx0_p[q3]], rows0_p[q3], semg[q3])
        cpb = pltpu.async_copy(table.at[idx1_p[q3]], rows1_p[c % 2], semg[q3])
        return cpa, cpb

    gathers = {0: start_gathers(0)}
    if N_CHUNKS > 1:
        gathers[1] = start_gathers(1)
    wbs = {}
    for c in range(N_CHUNKS):
        q3 = c % 3
        cpa, cpb = gathers.pop(c)
        cpa.wait()
        cpb.wait()
        rows0 = rows0_p[q3]
        rows1 = rows1_p[c % 2]

        def tok_body(t, carry):
            w = w0_v[pl.ds((c * CHUNK + t) * 16, 16)]
            for d in range(N_SLICES):
                sl = pl.ds(d * 16, 16)
                a = rows0[t, sl]
                b = rows1[t, sl]
                rows0[t, sl] = b + w * (a - b)
            return carry

        lax.fori_loop(0, CHUNK, tok_body, 0)
        wbs[c] = pltpu.async_copy(
            rows0, out.at[pl.ds(base_w + c * CHUNK, CHUNK)], semw[c % 2])
        if c + 2 < N_CHUNKS:
            if c - 1 in wbs:
                wbs.pop(c - 1).wait()
            gathers[c + 2] = start_gathers(c + 2)
    for c in sorted(wbs):
        wbs.pop(c).wait()


@functools.lru_cache(maxsize=1)
def _make_combine():
    return pl.kernel(
        _combine_body,
        out_type=jax.ShapeDtypeStruct((B, D_MODEL), jnp.float32),
        mesh=plsc.VectorSubcoreMesh(core_axis_name="c", subcore_axis_name="s"),
        scratch_types=[
            pltpu.VMEM((CHUNK,), jnp.int32),
            pltpu.VMEM((CHUNK,), jnp.int32),
            pltpu.VMEM((CHUNK,), jnp.int32),
            pltpu.VMEM((CHUNK,), jnp.int32),
            pltpu.VMEM((CHUNK,), jnp.int32),
            pltpu.VMEM((CHUNK,), jnp.int32),
            pltpu.VMEM((B_PER_W * 16,), jnp.float32),
            pltpu.VMEM((CHUNK, D_MODEL), jnp.float32),
            pltpu.VMEM((CHUNK, D_MODEL), jnp.float32),
            pltpu.VMEM((CHUNK, D_MODEL), jnp.float32),
            pltpu.VMEM((CHUNK, D_MODEL), jnp.float32),
            pltpu.VMEM((CHUNK, D_MODEL), jnp.float32),
            pltpu.SemaphoreType.DMA,
            pltpu.SemaphoreType.DMA,
            pltpu.SemaphoreType.DMA,
            pltpu.SemaphoreType.DMA,
            pltpu.SemaphoreType.DMA,
        ],
    )


def kernel(tier_outputs, query, W):
    logits, gate_weights, idx0, idx1, w0 = _router(query, W)
    table = tier_outputs.reshape(N_TIERS * B, D_MODEL)
    w0x = jnp.repeat(w0, 16)
    merged = _make_combine()(table, idx0, idx1, w0x)
    return merged, gate_weights, logits
